# Initial kernel scaffold; baseline (speedup 1.0000x reference)
import jax
import jax.numpy as jnp
from jax.experimental import pallas as pl


def kernel(x, edge_index, W1l, b1, W1r, W2l, b2, W2r):
    raise NotImplementedError("write your pallas kernel here")


# speedup vs baseline: 10.6857x; 10.6857x over previous
import functools

import jax
import jax.numpy as jnp
from jax import lax
from jax.experimental import pallas as pl
from jax.experimental.pallas import tpu as pltpu
from jax.experimental.pallas import tpu_sc as plsc

N = 10000
D = 128
H = 16
C = 40
E = 320000

NPAD = 10240
NC = 2
NS = 16
NW = NC * NS
CHUNK = 128
CHUNKS = 80
EPAD = NW * CHUNKS * CHUNK
RPT = NPAD // NS
RB = 512
GRID = NPAD // RB


def _mm1_body(x_ref, wl_ref, wr_ref, y1_ref, xr_ref):
    xb = x_ref[...]
    y1_ref[...] = jnp.dot(xb, wl_ref[...], preferred_element_type=jnp.float32)
    xr_ref[...] = jnp.dot(xb, wr_ref[...], preferred_element_type=jnp.float32)


def _mm1(x_pad, w1lt, w1rt):
    return pl.pallas_call(
        _mm1_body,
        grid=(GRID,),
        in_specs=[
            pl.BlockSpec((RB, D), lambda i: (i, 0)),
            pl.BlockSpec((D, H), lambda i: (0, 0)),
            pl.BlockSpec((D, H), lambda i: (0, 0)),
        ],
        out_specs=[
            pl.BlockSpec((RB, H), lambda i: (i, 0)),
            pl.BlockSpec((RB, H), lambda i: (i, 0)),
        ],
        out_shape=[
            jax.ShapeDtypeStruct((NPAD, H), jnp.float32),
            jax.ShapeDtypeStruct((NPAD, H), jnp.float32),
        ],
    )(x_pad, w1lt, w1rt)


def _make_seg(with_count):
    mesh = plsc.VectorSubcoreMesh(core_axis_name="c", subcore_axis_name="s")
    out_type = [jax.ShapeDtypeStruct((NC, NPAD, H), jnp.float32)]
    scratch = [
        pltpu.VMEM((CHUNKS, CHUNK), jnp.int32),
        pltpu.VMEM((CHUNKS, CHUNK), jnp.int32),
        pltpu.VMEM((CHUNK, H), jnp.float32),
        pltpu.VMEM_SHARED((NPAD, H), jnp.float32),
        pltpu.SemaphoreType.DMA,
    ]
    if with_count:
        out_type.append(jax.ShapeDtypeStruct((NC, NPAD, H), jnp.float32))
        scratch += [
            pltpu.VMEM((CHUNK, H), jnp.float32),
            pltpu.VMEM_SHARED((NPAD, H), jnp.float32),
        ]

    def body(table, src3, dst3, zagg, zcnt, ones, agg_out, cnt_out,
             src_v, dst_v, rows_v, sh_agg, sem, ones_v=None, sh_cnt=None):
        cid = lax.axis_index("c")
        sid = lax.axis_index("s")
        wid = sid * NC + cid
        r0 = sid * RPT
        pltpu.sync_copy(zagg.at[pl.ds(r0, RPT)], sh_agg.at[pl.ds(r0, RPT)])
        pltpu.sync_copy(src3.at[wid], src_v)
        pltpu.sync_copy(dst3.at[wid], dst_v)
        if with_count:
            pltpu.sync_copy(zcnt.at[pl.ds(r0, RPT)], sh_cnt.at[pl.ds(r0, RPT)])
            pltpu.sync_copy(ones, ones_v)
        plsc.subcore_barrier()

        def chunk_body(j, carry):
            pltpu.async_copy(table.at[src_v.at[j]], rows_v, sem).wait()
            pltpu.sync_copy(rows_v, sh_agg.at[dst_v.at[j]], add=True)
            if with_count:
                pltpu.sync_copy(ones_v, sh_cnt.at[dst_v.at[j]], add=True)
            return carry

        lax.fori_loop(0, CHUNKS, chunk_body, 0)
        plsc.subcore_barrier()
        pltpu.sync_copy(sh_agg.at[pl.ds(r0, RPT)],
                        agg_out.at[cid, pl.ds(r0, RPT)])
        if with_count:
            pltpu.sync_copy(sh_cnt.at[pl.ds(r0, RPT)],
                            cnt_out.at[cid, pl.ds(r0, RPT)])

    if with_count:
        def body_wc(table, src3, dst3, zagg, zcnt, ones, agg_out, cnt_out,
                    src_v, dst_v, rows_v, sh_agg, sem, ones_v, sh_cnt):
            body(table, src3, dst3, zagg, zcnt, ones, agg_out, cnt_out,
                 src_v, dst_v, rows_v, sh_agg, sem, ones_v, sh_cnt)
        fn = pl.kernel(body_wc, out_type=out_type, mesh=mesh,
                       scratch_types=scratch,
                       compiler_params=pltpu.CompilerParams(
                           use_tc_tiling_on_sc=False))

        def run(table, src3, dst3, zagg, zcnt, ones):
            return fn(table, src3, dst3, zagg, zcnt, ones)
    else:
        def body_nc(table, src3, dst3, zagg, agg_out,
                    src_v, dst_v, rows_v, sh_agg, sem):
            body(table, src3, dst3, zagg, None, None, agg_out, None,
                 src_v, dst_v, rows_v, sh_agg, sem)
        fn = pl.kernel(body_nc, out_type=out_type, mesh=mesh,
                       scratch_types=scratch,
                       compiler_params=pltpu.CompilerParams(
                           use_tc_tiling_on_sc=False))

        def run(table, src3, dst3, zagg, zcnt, ones):
            return fn(table, src3, dst3, zagg)
    return run


_seg_count = _make_seg(True)
_seg_plain = _make_seg(False)


def _act_body(agg_ref, cnt_ref, xr_ref, b1_ref, w_ref, h_ref, hr_ref):
    aggs = agg_ref[0] + agg_ref[1]
    cnts = cnt_ref[0] + cnt_ref[1]
    inv = 1.0 / jnp.maximum(cnts, 1.0)
    h = jnp.maximum(aggs * inv + b1_ref[...] + xr_ref[...], 0.0)
    h_ref[...] = h
    hr_ref[...] = jnp.dot(h, w_ref[...], preferred_element_type=jnp.float32)


def _act(agg, cnt, xr, b1r, w2rt):
    return pl.pallas_call(
        _act_body,
        grid=(GRID,),
        in_specs=[
            pl.BlockSpec((NC, RB, H), lambda i: (0, i, 0)),
            pl.BlockSpec((NC, RB, H), lambda i: (0, i, 0)),
            pl.BlockSpec((RB, H), lambda i: (i, 0)),
            pl.BlockSpec((1, H), lambda i: (0, 0)),
            pl.BlockSpec((H, C), lambda i: (0, 0)),
        ],
        out_specs=[
            pl.BlockSpec((RB, H), lambda i: (i, 0)),
            pl.BlockSpec((RB, C), lambda i: (i, 0)),
        ],
        out_shape=[
            jax.ShapeDtypeStruct((NPAD, H), jnp.float32),
            jax.ShapeDtypeStruct((NPAD, C), jnp.float32),
        ],
    )(agg, cnt, xr, b1r, w2rt)


def _out_body(agg_ref, cnt_ref, hr_ref, b2_ref, w_ref, o_ref):
    aggs = agg_ref[0] + agg_ref[1]
    cnts = cnt_ref[0] + cnt_ref[1]
    mean2 = aggs * (1.0 / jnp.maximum(cnts, 1.0))
    o = (jnp.dot(mean2, w_ref[...], preferred_element_type=jnp.float32)
         + b2_ref[...] + hr_ref[...])
    m = jnp.max(o, axis=1, keepdims=True)
    e = jnp.exp(o - m)
    s = jnp.sum(e, axis=1, keepdims=True)
    o_ref[...] = o - m - jnp.log(s)


def _outk(agg2, cnt, hr2, b2r, w2lt):
    return pl.pallas_call(
        _out_body,
        grid=(GRID,),
        in_specs=[
            pl.BlockSpec((NC, RB, H), lambda i: (0, i, 0)),
            pl.BlockSpec((NC, RB, H), lambda i: (0, i, 0)),
            pl.BlockSpec((RB, C), lambda i: (i, 0)),
            pl.BlockSpec((1, C), lambda i: (0, 0)),
            pl.BlockSpec((H, C), lambda i: (0, 0)),
        ],
        out_specs=pl.BlockSpec((RB, C), lambda i: (i, 0)),
        out_shape=jax.ShapeDtypeStruct((NPAD, C), jnp.float32),
    )(agg2, cnt, hr2, b2r, w2lt)


@jax.jit
def kernel(x, edge_index, W1l, b1, W1r, W2l, b2, W2r):
    x_pad = jnp.zeros((NPAD, D), jnp.float32).at[:N].set(x)
    src = jnp.concatenate(
        [edge_index[0], jnp.zeros((EPAD - E,), jnp.int32)]).reshape(
            NW, CHUNKS, CHUNK)
    dst = jnp.concatenate(
        [edge_index[1], jnp.full((EPAD - E,), N, jnp.int32)]).reshape(
            NW, CHUNKS, CHUNK)
    zagg = jnp.zeros((NPAD, H), jnp.float32)
    zcnt = jnp.zeros((NPAD, H), jnp.float32)
    ones = jnp.ones((CHUNK, H), jnp.float32)

    y1, xr = _mm1(x_pad, W1l.T, W1r.T)
    agg1, cnt = _seg_count(y1, src, dst, zagg, zcnt, ones)
    h, hr2 = _act(agg1, cnt, xr, b1.reshape(1, H), W2r.T)

    (agg2,) = _seg_plain(h, src, dst, zagg, zcnt, ones)
    out = _outk(agg2, cnt, hr2, b2.reshape(1, C), W2l.T)
    return out[:N]


# speedup vs baseline: 14.3002x; 1.3383x over previous
import functools

import jax
import jax.numpy as jnp
from jax import lax
from jax.experimental import pallas as pl
from jax.experimental.pallas import tpu as pltpu
from jax.experimental.pallas import tpu_sc as plsc

N = 10000
D = 128
H = 16
C = 40
E = 320000

NPAD = 10240
NC = 2
NS = 16
NW = NC * NS
CHUNK = 128
CHUNKS = 80
EPAD = NW * CHUNKS * CHUNK
RPT = NPAD // NS
RB = 512
GRID = NPAD // RB


def _mm1_body(x_ref, wl_ref, wr_ref, y1_ref, xr_ref):
    xb = x_ref[...]
    y1_ref[...] = jnp.dot(xb, wl_ref[...], preferred_element_type=jnp.float32)
    xr_ref[...] = jnp.dot(xb, wr_ref[...], preferred_element_type=jnp.float32)


def _mm1(x_pad, w1lt, w1rt):
    return pl.pallas_call(
        _mm1_body,
        grid=(GRID,),
        in_specs=[
            pl.BlockSpec((RB, D), lambda i: (i, 0)),
            pl.BlockSpec((D, H), lambda i: (0, 0)),
            pl.BlockSpec((D, H), lambda i: (0, 0)),
        ],
        out_specs=[
            pl.BlockSpec((RB, H), lambda i: (i, 0)),
            pl.BlockSpec((RB, H), lambda i: (i, 0)),
        ],
        out_shape=[
            jax.ShapeDtypeStruct((NPAD, H), jnp.float32),
            jax.ShapeDtypeStruct((NPAD, H), jnp.float32),
        ],
    )(x_pad, w1lt, w1rt)


def _make_seg(with_count):
    mesh = plsc.VectorSubcoreMesh(core_axis_name="c", subcore_axis_name="s")
    out_type = [jax.ShapeDtypeStruct((NC, NPAD, H), jnp.float32)]
    scratch = [
        pltpu.VMEM((CHUNKS, CHUNK), jnp.int32),
        pltpu.VMEM((CHUNKS, CHUNK), jnp.int32),
        pltpu.VMEM((CHUNK, H), jnp.float32),
        pltpu.VMEM((CHUNK, H), jnp.float32),
        pltpu.VMEM_SHARED((NPAD, H), jnp.float32),
        pltpu.SemaphoreType.DMA,
        pltpu.SemaphoreType.DMA,
        pltpu.SemaphoreType.DMA,
        pltpu.SemaphoreType.DMA,
    ]
    if with_count:
        out_type.append(jax.ShapeDtypeStruct((NC, NPAD, H), jnp.float32))
        scratch += [
            pltpu.VMEM((CHUNK, H), jnp.float32),
            pltpu.VMEM_SHARED((NPAD, H), jnp.float32),
            pltpu.SemaphoreType.DMA,
            pltpu.SemaphoreType.DMA,
        ]

    def body(table, src3, dst3, zagg, zcnt, ones, agg_out, cnt_out,
             src_v, dst_v, rows0, rows1, sh_agg, g0, g1, s0, s1,
             ones_v=None, sh_cnt=None, c0=None, c1=None):
        cid = lax.axis_index("c")
        sid = lax.axis_index("s")
        wid = sid * NC + cid
        r0 = sid * RPT
        pltpu.sync_copy(zagg.at[pl.ds(r0, RPT)], sh_agg.at[pl.ds(r0, RPT)])
        pltpu.sync_copy(src3.at[wid], src_v)
        pltpu.sync_copy(dst3.at[wid], dst_v)
        if with_count:
            pltpu.sync_copy(zcnt.at[pl.ds(r0, RPT)], sh_cnt.at[pl.ds(r0, RPT)])
            pltpu.sync_copy(ones, ones_v)
        plsc.subcore_barrier()

        bufs = ((rows0, g0, s0, c0), (rows1, g1, s1, c1))

        pltpu.async_copy(table.at[src_v.at[0]], rows0, g0)
        pltpu.async_copy(table.at[src_v.at[1]], rows1, g1)

        def pair_body(i, carry):
            for b, (buf, gsem, ssem, csem) in enumerate(bufs):
                c = 2 * i + b
                pltpu.make_async_copy(table.at[src_v.at[c]], buf, gsem).wait()
                sc = pltpu.async_copy(buf, sh_agg.at[dst_v.at[c]], ssem,
                                      add=True)
                if with_count:
                    cc = pltpu.async_copy(ones_v, sh_cnt.at[dst_v.at[c]],
                                          csem, add=True)
                sc.wait()

                @pl.when(i < CHUNKS // 2 - 1)
                def _():
                    pltpu.async_copy(table.at[src_v.at[c + 2]], buf, gsem)

                if with_count:
                    cc.wait()
            return carry

        lax.fori_loop(0, CHUNKS // 2, pair_body, 0)
        plsc.subcore_barrier()
        pltpu.sync_copy(sh_agg.at[pl.ds(r0, RPT)],
                        agg_out.at[cid, pl.ds(r0, RPT)])
        if with_count:
            pltpu.sync_copy(sh_cnt.at[pl.ds(r0, RPT)],
                            cnt_out.at[cid, pl.ds(r0, RPT)])

    if with_count:
        def body_wc(table, src3, dst3, zagg, zcnt, ones, agg_out, cnt_out,
                    src_v, dst_v, rows0, rows1, sh_agg, g0, g1, s0, s1,
                    ones_v, sh_cnt, c0, c1):
            body(table, src3, dst3, zagg, zcnt, ones, agg_out, cnt_out,
                 src_v, dst_v, rows0, rows1, sh_agg, g0, g1, s0, s1,
                 ones_v, sh_cnt, c0, c1)
        fn = pl.kernel(body_wc, out_type=out_type, mesh=mesh,
                       scratch_types=scratch,
                       compiler_params=pltpu.CompilerParams(
                           use_tc_tiling_on_sc=False))

        def run(table, src3, dst3, zagg, zcnt, ones):
            return fn(table, src3, dst3, zagg, zcnt, ones)
    else:
        def body_nc(table, src3, dst3, zagg, agg_out,
                    src_v, dst_v, rows0, rows1, sh_agg, g0, g1, s0, s1):
            body(table, src3, dst3, zagg, None, None, agg_out, None,
                 src_v, dst_v, rows0, rows1, sh_agg, g0, g1, s0, s1)
        fn = pl.kernel(body_nc, out_type=out_type, mesh=mesh,
                       scratch_types=scratch,
                       compiler_params=pltpu.CompilerParams(
                           use_tc_tiling_on_sc=False))

        def run(table, src3, dst3, zagg, zcnt, ones):
            return fn(table, src3, dst3, zagg)
    return run


_seg_count = _make_seg(True)
_seg_plain = _make_seg(False)


def _act_body(agg_ref, cnt_ref, xr_ref, b1_ref, w_ref, h_ref, hr_ref):
    aggs = agg_ref[0] + agg_ref[1]
    cnts = cnt_ref[0] + cnt_ref[1]
    inv = 1.0 / jnp.maximum(cnts, 1.0)
    h = jnp.maximum(aggs * inv + b1_ref[...] + xr_ref[...], 0.0)
    h_ref[...] = h
    hr_ref[...] = jnp.dot(h, w_ref[...], preferred_element_type=jnp.float32)


def _act(agg, cnt, xr, b1r, w2rt):
    return pl.pallas_call(
        _act_body,
        grid=(GRID,),
        in_specs=[
            pl.BlockSpec((NC, RB, H), lambda i: (0, i, 0)),
            pl.BlockSpec((NC, RB, H), lambda i: (0, i, 0)),
            pl.BlockSpec((RB, H), lambda i: (i, 0)),
            pl.BlockSpec((1, H), lambda i: (0, 0)),
            pl.BlockSpec((H, C), lambda i: (0, 0)),
        ],
        out_specs=[
            pl.BlockSpec((RB, H), lambda i: (i, 0)),
            pl.BlockSpec((RB, C), lambda i: (i, 0)),
        ],
        out_shape=[
            jax.ShapeDtypeStruct((NPAD, H), jnp.float32),
            jax.ShapeDtypeStruct((NPAD, C), jnp.float32),
        ],
    )(agg, cnt, xr, b1r, w2rt)


def _out_body(agg_ref, cnt_ref, hr_ref, b2_ref, w_ref, o_ref):
    aggs = agg_ref[0] + agg_ref[1]
    cnts = cnt_ref[0] + cnt_ref[1]
    mean2 = aggs * (1.0 / jnp.maximum(cnts, 1.0))
    o = (jnp.dot(mean2, w_ref[...], preferred_element_type=jnp.float32)
         + b2_ref[...] + hr_ref[...])
    m = jnp.max(o, axis=1, keepdims=True)
    e = jnp.exp(o - m)
    s = jnp.sum(e, axis=1, keepdims=True)
    o_ref[...] = o - m - jnp.log(s)


def _outk(agg2, cnt, hr2, b2r, w2lt):
    return pl.pallas_call(
        _out_body,
        grid=(GRID,),
        in_specs=[
            pl.BlockSpec((NC, RB, H), lambda i: (0, i, 0)),
            pl.BlockSpec((NC, RB, H), lambda i: (0, i, 0)),
            pl.BlockSpec((RB, C), lambda i: (i, 0)),
            pl.BlockSpec((1, C), lambda i: (0, 0)),
            pl.BlockSpec((H, C), lambda i: (0, 0)),
        ],
        out_specs=pl.BlockSpec((RB, C), lambda i: (i, 0)),
        out_shape=jax.ShapeDtypeStruct((NPAD, C), jnp.float32),
    )(agg2, cnt, hr2, b2r, w2lt)


@jax.jit
def kernel(x, edge_index, W1l, b1, W1r, W2l, b2, W2r):
    x_pad = jnp.zeros((NPAD, D), jnp.float32).at[:N].set(x)
    src = jnp.concatenate(
        [edge_index[0], jnp.zeros((EPAD - E,), jnp.int32)]).reshape(
            NW, CHUNKS, CHUNK)
    dst = jnp.concatenate(
        [edge_index[1], jnp.full((EPAD - E,), N, jnp.int32)]).reshape(
            NW, CHUNKS, CHUNK)
    zagg = jnp.zeros((NPAD, H), jnp.float32)
    zcnt = jnp.zeros((NPAD, H), jnp.float32)
    ones = jnp.ones((CHUNK, H), jnp.float32)

    y1, xr = _mm1(x_pad, W1l.T, W1r.T)
    agg1, cnt = _seg_count(y1, src, dst, zagg, zcnt, ones)
    h, hr2 = _act(agg1, cnt, xr, b1.reshape(1, H), W2r.T)

    (agg2,) = _seg_plain(h, src, dst, zagg, zcnt, ones)
    out = _outk(agg2, cnt, hr2, b2.reshape(1, C), W2l.T)
    return out[:N]


# speedup vs baseline: 19.7451x; 1.3808x over previous
import jax
import jax.numpy as jnp
from jax import lax
from jax.experimental import pallas as pl
from jax.experimental.pallas import tpu as pltpu
from jax.experimental.pallas import tpu_sc as plsc

N = 10000
D = 128
H = 16
C = 40
E = 320000

NPAD = 10240
NC = 2
NS = 16
NW = NC * NS
CHUNK = 128
TOTCH = E // CHUNK
BASECH = TOTCH // NW
EXTRA = TOTCH - BASECH * NW
RPT = NPAD // NS


def _mm1_body(x_ref, wl_ref, wr_ref, y1_ref, xr_ref):
    xb = x_ref[...]
    y1_ref[...] = jnp.dot(xb, wl_ref[...], preferred_element_type=jnp.float32)
    xr_ref[...] = jnp.dot(xb, wr_ref[...], preferred_element_type=jnp.float32)


def _mm1(x, w1lt, w1rt):
    return pl.pallas_call(
        _mm1_body,
        out_shape=[
            jax.ShapeDtypeStruct((N, H), jnp.float32),
            jax.ShapeDtypeStruct((N, H), jnp.float32),
        ],
    )(x, w1lt, w1rt)


def _make_seg(with_count):
    mesh = plsc.VectorSubcoreMesh(core_axis_name="c", subcore_axis_name="s")
    out_type = [jax.ShapeDtypeStruct((NC, NPAD, H), jnp.float32)]
    scratch = [
        pltpu.VMEM((BASECH + 1, CHUNK), jnp.int32),
        pltpu.VMEM((BASECH + 1, CHUNK), jnp.int32),
        pltpu.VMEM((CHUNK, H), jnp.float32),
        pltpu.VMEM((CHUNK, H), jnp.float32),
        pltpu.VMEM_SHARED((NPAD, H), jnp.float32),
        pltpu.SemaphoreType.DMA,
        pltpu.SemaphoreType.DMA,
        pltpu.SemaphoreType.DMA,
        pltpu.SemaphoreType.DMA,
    ]
    if with_count:
        out_type.append(jax.ShapeDtypeStruct((NC, NPAD, H), jnp.float32))
        scratch += [
            pltpu.VMEM((CHUNK, H), jnp.float32),
            pltpu.VMEM_SHARED((NPAD, H), jnp.float32),
            pltpu.SemaphoreType.DMA,
            pltpu.SemaphoreType.DMA,
        ]

    def body(table, ei3, zagg, zcnt, ones, agg_out, cnt_out,
             src_v, dst_v, rows0, rows1, sh_agg, g0, g1, s0, s1,
             ones_v=None, sh_cnt=None, c0=None, c1=None):
        cid = lax.axis_index("c")
        sid = lax.axis_index("s")
        wid = sid * NC + cid
        r0 = sid * RPT
        pltpu.sync_copy(zagg.at[pl.ds(r0, RPT)], sh_agg.at[pl.ds(r0, RPT)])
        base = wid * BASECH
        pltpu.sync_copy(ei3.at[0, pl.ds(base, BASECH)],
                        src_v.at[pl.ds(0, BASECH)])
        pltpu.sync_copy(ei3.at[1, pl.ds(base, BASECH)],
                        dst_v.at[pl.ds(0, BASECH)])

        @pl.when(wid < EXTRA)
        def _():
            pltpu.sync_copy(ei3.at[0, BASECH * NW + wid], src_v.at[BASECH])
            pltpu.sync_copy(ei3.at[1, BASECH * NW + wid], dst_v.at[BASECH])

        if with_count:
            pltpu.sync_copy(zcnt.at[pl.ds(r0, RPT)], sh_cnt.at[pl.ds(r0, RPT)])
            pltpu.sync_copy(ones, ones_v)
        plsc.subcore_barrier()

        bufs = ((rows0, g0, s0, c0), (rows1, g1, s1, c1))

        pltpu.async_copy(table.at[src_v.at[0]], rows0, g0)
        pltpu.async_copy(table.at[src_v.at[1]], rows1, g1)

        def pair_body(i, carry):
            for b, (buf, gsem, ssem, csem) in enumerate(bufs):
                c = 2 * i + b
                pltpu.make_async_copy(table.at[src_v.at[c]], buf, gsem).wait()
                sc = pltpu.async_copy(buf, sh_agg.at[dst_v.at[c]], ssem,
                                      add=True)
                if with_count:
                    cc = pltpu.async_copy(ones_v, sh_cnt.at[dst_v.at[c]],
                                          csem, add=True)
                sc.wait()

                @pl.when(i < BASECH // 2 - 1)
                def _():
                    pltpu.async_copy(table.at[src_v.at[c + 2]], buf, gsem)

                if with_count:
                    cc.wait()
            return carry

        lax.fori_loop(0, BASECH // 2, pair_body, 0)

        @pl.when(wid < EXTRA)
        def _():
            pltpu.async_copy(table.at[src_v.at[BASECH]], rows0, g0).wait()
            sc = pltpu.async_copy(rows0, sh_agg.at[dst_v.at[BASECH]], s0,
                                  add=True)
            if with_count:
                pltpu.async_copy(ones_v, sh_cnt.at[dst_v.at[BASECH]], c0,
                                 add=True).wait()
            sc.wait()

        plsc.subcore_barrier()
        pltpu.sync_copy(sh_agg.at[pl.ds(r0, RPT)],
                        agg_out.at[cid, pl.ds(r0, RPT)])
        if with_count:
            pltpu.sync_copy(sh_cnt.at[pl.ds(r0, RPT)],
                            cnt_out.at[cid, pl.ds(r0, RPT)])

    if with_count:
        def body_wc(table, ei3, zagg, zcnt, ones, agg_out, cnt_out,
                    src_v, dst_v, rows0, rows1, sh_agg, g0, g1, s0, s1,
                    ones_v, sh_cnt, c0, c1):
            body(table, ei3, zagg, zcnt, ones, agg_out, cnt_out,
                 src_v, dst_v, rows0, rows1, sh_agg, g0, g1, s0, s1,
                 ones_v, sh_cnt, c0, c1)
        fn = pl.kernel(body_wc, out_type=out_type, mesh=mesh,
                       scratch_types=scratch,
                       compiler_params=pltpu.CompilerParams(
                           use_tc_tiling_on_sc=False))

        def run(table, ei3, zagg, zcnt, ones):
            return fn(table, ei3, zagg, zcnt, ones)
    else:
        def body_nc(table, ei3, zagg, agg_out,
                    src_v, dst_v, rows0, rows1, sh_agg, g0, g1, s0, s1):
            body(table, ei3, zagg, None, None, agg_out, None,
                 src_v, dst_v, rows0, rows1, sh_agg, g0, g1, s0, s1)
        fn = pl.kernel(body_nc, out_type=out_type, mesh=mesh,
                       scratch_types=scratch,
                       compiler_params=pltpu.CompilerParams(
                           use_tc_tiling_on_sc=False))

        def run(table, ei3, zagg, zcnt, ones):
            return fn(table, ei3, zagg)
    return run


_seg_count = _make_seg(True)
_seg_plain = _make_seg(False)


def _act_body(agg_ref, cnt_ref, xr_ref, b1_ref, w_ref, h_ref, hr_ref):
    aggs = agg_ref[0, :N] + agg_ref[1, :N]
    cnts = cnt_ref[0, :N] + cnt_ref[1, :N]
    inv = 1.0 / jnp.maximum(cnts, 1.0)
    h = jnp.maximum(aggs * inv + b1_ref[...] + xr_ref[...], 0.0)
    h_ref[...] = h
    hr_ref[...] = jnp.dot(h, w_ref[...], preferred_element_type=jnp.float32)


def _act(agg, cnt, xr, b1r, w2rt):
    return pl.pallas_call(
        _act_body,
        out_shape=[
            jax.ShapeDtypeStruct((N, H), jnp.float32),
            jax.ShapeDtypeStruct((N, C), jnp.float32),
        ],
    )(agg, cnt, xr, b1r, w2rt)


def _out_body(agg_ref, cnt_ref, hr_ref, b2_ref, w_ref, o_ref):
    aggs = agg_ref[0, :N] + agg_ref[1, :N]
    cnts = cnt_ref[0, :N] + cnt_ref[1, :N]
    mean2 = aggs * (1.0 / jnp.maximum(cnts, 1.0))
    o = (jnp.dot(mean2, w_ref[...], preferred_element_type=jnp.float32)
         + b2_ref[...] + hr_ref[...])
    m = jnp.max(o, axis=1, keepdims=True)
    e = jnp.exp(o - m)
    s = jnp.sum(e, axis=1, keepdims=True)
    o_ref[...] = o - m - jnp.log(s)


def _outk(agg2, cnt, hr2, b2r, w2lt):
    return pl.pallas_call(
        _out_body,
        out_shape=jax.ShapeDtypeStruct((N, C), jnp.float32),
    )(agg2, cnt, hr2, b2r, w2lt)


@jax.jit
def kernel(x, edge_index, W1l, b1, W1r, W2l, b2, W2r):
    ei3 = edge_index.reshape(2, TOTCH, CHUNK)
    zagg = jnp.zeros((NPAD, H), jnp.float32)
    zcnt = jnp.zeros((NPAD, H), jnp.float32)
    ones = jnp.ones((CHUNK, H), jnp.float32)

    y1, xr = _mm1(x, W1l.T, W1r.T)
    agg1, cnt = _seg_count(y1, ei3, zagg, zcnt, ones)
    h, hr2 = _act(agg1, cnt, xr, b1.reshape(1, H), W2r.T)

    (agg2,) = _seg_plain(h, ei3, zagg, zcnt, ones)
    return _outk(agg2, cnt, hr2, b2.reshape(1, C), W2l.T)


# speedup vs baseline: 22.3245x; 1.1306x over previous
import jax
import jax.numpy as jnp
from jax import lax
from jax.experimental import pallas as pl
from jax.experimental.pallas import tpu as pltpu
from jax.experimental.pallas import tpu_sc as plsc

N = 10000
D = 128
H = 16
C = 40
E = 320000

NPAD = 10240
NC = 2
NS = 16
NW = NC * NS
CHUNK = 128
TOTCH = E // CHUNK
BASECH = TOTCH // NW
EXTRA = TOTCH - BASECH * NW
RPT = NPAD // NS


def _mm1_body(x_ref, wl_ref, wr_ref, y1_ref, xr_ref):
    xb = x_ref[...]
    y1_ref[...] = jnp.dot(xb, wl_ref[...], preferred_element_type=jnp.float32)
    xr_ref[...] = jnp.dot(xb, wr_ref[...], preferred_element_type=jnp.float32)


def _mm1(x, w1lt, w1rt):
    return pl.pallas_call(
        _mm1_body,
        out_shape=[
            jax.ShapeDtypeStruct((N, H), jnp.float32),
            jax.ShapeDtypeStruct((N, H), jnp.float32),
        ],
    )(x, w1lt, w1rt)


def _make_seg(with_count):
    mesh = plsc.VectorSubcoreMesh(core_axis_name="c", subcore_axis_name="s")
    out_type = [jax.ShapeDtypeStruct((NC, NPAD, H), jnp.float32)]
    scratch = [
        pltpu.VMEM((BASECH + 1, CHUNK), jnp.int32),
        pltpu.VMEM((BASECH + 1, CHUNK), jnp.int32),
        pltpu.VMEM((CHUNK, H), jnp.float32),
        pltpu.VMEM((CHUNK, H), jnp.float32),
        pltpu.VMEM((CHUNK, H), jnp.float32),
        pltpu.VMEM_SHARED((NPAD, H), jnp.float32),
        pltpu.SemaphoreType.DMA,
        pltpu.SemaphoreType.DMA,
        pltpu.SemaphoreType.DMA,
        pltpu.SemaphoreType.DMA,
        pltpu.SemaphoreType.DMA,
        pltpu.SemaphoreType.DMA,
    ]
    if with_count:
        out_type.append(jax.ShapeDtypeStruct((NC, NPAD, H), jnp.float32))
        scratch += [
            pltpu.VMEM((CHUNK, H), jnp.float32),
            pltpu.VMEM_SHARED((NPAD, H), jnp.float32),
            pltpu.SemaphoreType.DMA,
            pltpu.SemaphoreType.DMA,
            pltpu.SemaphoreType.DMA,
        ]

    def body(table, ei3, zagg, zcnt, ones, agg_out, cnt_out,
             src_v, dst_v, rows0, rows1, rows2, sh_agg, g0, g1, g2,
             s0, s1, s2, ones_v=None, sh_cnt=None, c0=None, c1=None, c2=None):
        cid = lax.axis_index("c")
        sid = lax.axis_index("s")
        wid = sid * NC + cid
        r0 = sid * RPT
        pltpu.sync_copy(zagg.at[pl.ds(r0, RPT)], sh_agg.at[pl.ds(r0, RPT)])
        base = wid * BASECH
        pltpu.sync_copy(ei3.at[0, pl.ds(base, BASECH)],
                        src_v.at[pl.ds(0, BASECH)])
        pltpu.sync_copy(ei3.at[1, pl.ds(base, BASECH)],
                        dst_v.at[pl.ds(0, BASECH)])

        @pl.when(wid < EXTRA)
        def _():
            pltpu.sync_copy(ei3.at[0, BASECH * NW + wid], src_v.at[BASECH])
            pltpu.sync_copy(ei3.at[1, BASECH * NW + wid], dst_v.at[BASECH])

        if with_count:
            pltpu.sync_copy(zcnt.at[pl.ds(r0, RPT)], sh_cnt.at[pl.ds(r0, RPT)])
            pltpu.sync_copy(ones, ones_v)
        plsc.subcore_barrier()

        bufs = ((rows0, g0, s0, c0), (rows1, g1, s1, c1),
                (rows2, g2, s2, c2))
        NB = len(bufs)

        pltpu.async_copy(table.at[src_v.at[0]], rows0, g0)
        pltpu.async_copy(table.at[src_v.at[1]], rows1, g1)
        pltpu.async_copy(table.at[src_v.at[2]], rows2, g2)

        def pair_body(i, carry):
            for b, (buf, gsem, ssem, csem) in enumerate(bufs):
                c = NB * i + b
                pltpu.make_async_copy(table.at[src_v.at[c]], buf, gsem).wait()
                sc = pltpu.async_copy(buf, sh_agg.at[dst_v.at[c]], ssem,
                                      add=True)
                if with_count:
                    cc = pltpu.async_copy(ones_v, sh_cnt.at[dst_v.at[c]],
                                          csem, add=True)
                sc.wait()

                @pl.when(i < BASECH // NB - 1)
                def _():
                    pltpu.async_copy(table.at[src_v.at[c + NB]], buf, gsem)

                if with_count:
                    cc.wait()
            return carry

        lax.fori_loop(0, BASECH // NB, pair_body, 0)

        @pl.when(wid < EXTRA)
        def _():
            pltpu.async_copy(table.at[src_v.at[BASECH]], rows0, g0).wait()
            sc = pltpu.async_copy(rows0, sh_agg.at[dst_v.at[BASECH]], s0,
                                  add=True)
            if with_count:
                pltpu.async_copy(ones_v, sh_cnt.at[dst_v.at[BASECH]], c0,
                                 add=True).wait()
            sc.wait()

        plsc.subcore_barrier()
        pltpu.sync_copy(sh_agg.at[pl.ds(r0, RPT)],
                        agg_out.at[cid, pl.ds(r0, RPT)])
        if with_count:
            pltpu.sync_copy(sh_cnt.at[pl.ds(r0, RPT)],
                            cnt_out.at[cid, pl.ds(r0, RPT)])

    if with_count:
        def body_wc(table, ei3, zagg, zcnt, ones, agg_out, cnt_out,
                    src_v, dst_v, rows0, rows1, rows2, sh_agg, g0, g1, g2,
                    s0, s1, s2, ones_v, sh_cnt, c0, c1, c2):
            body(table, ei3, zagg, zcnt, ones, agg_out, cnt_out,
                 src_v, dst_v, rows0, rows1, rows2, sh_agg, g0, g1, g2,
                 s0, s1, s2, ones_v, sh_cnt, c0, c1, c2)
        fn = pl.kernel(body_wc, out_type=out_type, mesh=mesh,
                       scratch_types=scratch,
                       compiler_params=pltpu.CompilerParams(
                           use_tc_tiling_on_sc=False))

        def run(table, ei3, zagg, zcnt, ones):
            return fn(table, ei3, zagg, zcnt, ones)
    else:
        def body_nc(table, ei3, zagg, agg_out,
                    src_v, dst_v, rows0, rows1, rows2, sh_agg, g0, g1, g2,
                    s0, s1, s2):
            body(table, ei3, zagg, None, None, agg_out, None,
                 src_v, dst_v, rows0, rows1, rows2, sh_agg, g0, g1, g2,
                 s0, s1, s2)
        fn = pl.kernel(body_nc, out_type=out_type, mesh=mesh,
                       scratch_types=scratch,
                       compiler_params=pltpu.CompilerParams(
                           use_tc_tiling_on_sc=False))

        def run(table, ei3, zagg, zcnt, ones):
            return fn(table, ei3, zagg)
    return run


_seg_count = _make_seg(True)
_seg_plain = _make_seg(False)


def _act_body(agg_ref, cnt_ref, xr_ref, b1_ref, w_ref, h_ref, hr_ref):
    aggs = agg_ref[0, :N] + agg_ref[1, :N]
    cnts = cnt_ref[0, :N] + cnt_ref[1, :N]
    inv = 1.0 / jnp.maximum(cnts, 1.0)
    h = jnp.maximum(aggs * inv + b1_ref[...] + xr_ref[...], 0.0)
    h_ref[...] = h
    hr_ref[...] = jnp.dot(h, w_ref[...], preferred_element_type=jnp.float32)


def _act(agg, cnt, xr, b1r, w2rt):
    return pl.pallas_call(
        _act_body,
        out_shape=[
            jax.ShapeDtypeStruct((N, H), jnp.float32),
            jax.ShapeDtypeStruct((N, C), jnp.float32),
        ],
    )(agg, cnt, xr, b1r, w2rt)


def _out_body(agg_ref, cnt_ref, hr_ref, b2_ref, w_ref, o_ref):
    aggs = agg_ref[0, :N] + agg_ref[1, :N]
    cnts = cnt_ref[0, :N] + cnt_ref[1, :N]
    mean2 = aggs * (1.0 / jnp.maximum(cnts, 1.0))
    o = (jnp.dot(mean2, w_ref[...], preferred_element_type=jnp.float32)
         + b2_ref[...] + hr_ref[...])
    m = jnp.max(o, axis=1, keepdims=True)
    e = jnp.exp(o - m)
    s = jnp.sum(e, axis=1, keepdims=True)
    o_ref[...] = o - m - jnp.log(s)


def _outk(agg2, cnt, hr2, b2r, w2lt):
    return pl.pallas_call(
        _out_body,
        out_shape=jax.ShapeDtypeStruct((N, C), jnp.float32),
    )(agg2, cnt, hr2, b2r, w2lt)


@jax.jit
def kernel(x, edge_index, W1l, b1, W1r, W2l, b2, W2r):
    ei3 = edge_index.reshape(2, TOTCH, CHUNK)
    zagg = jnp.zeros((NPAD, H), jnp.float32)
    zcnt = jnp.zeros((NPAD, H), jnp.float32)
    ones = jnp.ones((CHUNK, H), jnp.float32)

    y1, xr = _mm1(x, W1l.T, W1r.T)
    agg1, cnt = _seg_count(y1, ei3, zagg, zcnt, ones)
    h, hr2 = _act(agg1, cnt, xr, b1.reshape(1, H), W2r.T)

    (agg2,) = _seg_plain(h, ei3, zagg, zcnt, ones)
    return _outk(agg2, cnt, hr2, b2.reshape(1, C), W2l.T)


# speedup vs baseline: 28.9586x; 1.2972x over previous
import jax
import jax.numpy as jnp
from jax import lax
from jax.experimental import pallas as pl
from jax.experimental.pallas import tpu as pltpu
from jax.experimental.pallas import tpu_sc as plsc

N = 10000
D = 128
H = 16
C = 40
E = 320000

NPAD = 10240
NC = 2
NS = 16
NW = NC * NS
CHUNK = 128
TOTCH = E // CHUNK
BASECH = TOTCH // NW
EXTRA = TOTCH - BASECH * NW
RPT = NPAD // NS


P = N // 8
PL = 8 * H
PC = 8 * C


def _mm1_body(x_ref, wb_ref, y1_ref, xr_ref):
    o = jnp.dot(x_ref[...], wb_ref[...], preferred_element_type=jnp.float32)
    y1_ref[...] = o[:, :PL]
    xr_ref[...] = o[:, PL:]


def _mm1(x1024, w1big):
    return pl.pallas_call(
        _mm1_body,
        out_shape=[
            jax.ShapeDtypeStruct((P, PL), jnp.float32),
            jax.ShapeDtypeStruct((P, PL), jnp.float32),
        ],
    )(x1024, w1big)


def _make_seg(with_count):
    mesh = plsc.VectorSubcoreMesh(core_axis_name="c", subcore_axis_name="s")
    out_type = [jax.ShapeDtypeStruct((NC, NPAD, H), jnp.float32)]
    scratch = [
        pltpu.VMEM((BASECH + 1, CHUNK), jnp.int32),
        pltpu.VMEM((BASECH + 1, CHUNK), jnp.int32),
        pltpu.VMEM((CHUNK, H), jnp.float32),
        pltpu.VMEM((CHUNK, H), jnp.float32),
        pltpu.VMEM((CHUNK, H), jnp.float32),
        pltpu.VMEM_SHARED((NPAD, H), jnp.float32),
        pltpu.SemaphoreType.DMA,
        pltpu.SemaphoreType.DMA,
        pltpu.SemaphoreType.DMA,
        pltpu.SemaphoreType.DMA,
        pltpu.SemaphoreType.DMA,
        pltpu.SemaphoreType.DMA,
    ]
    if with_count:
        out_type.append(jax.ShapeDtypeStruct((NC, NPAD, H), jnp.float32))
        scratch += [
            pltpu.VMEM((CHUNK, H), jnp.float32),
            pltpu.VMEM_SHARED((NPAD, H), jnp.float32),
            pltpu.SemaphoreType.DMA,
            pltpu.SemaphoreType.DMA,
            pltpu.SemaphoreType.DMA,
        ]

    def body(table, ei3, zagg, zcnt, ones, agg_out, cnt_out,
             src_v, dst_v, rows0, rows1, rows2, sh_agg, g0, g1, g2,
             s0, s1, s2, ones_v=None, sh_cnt=None, c0=None, c1=None, c2=None):
        cid = lax.axis_index("c")
        sid = lax.axis_index("s")
        wid = sid * NC + cid
        r0 = sid * RPT
        pltpu.sync_copy(zagg.at[pl.ds(r0, RPT)], sh_agg.at[pl.ds(r0, RPT)])
        base = wid * BASECH
        pltpu.sync_copy(ei3.at[0, pl.ds(base, BASECH)],
                        src_v.at[pl.ds(0, BASECH)])
        pltpu.sync_copy(ei3.at[1, pl.ds(base, BASECH)],
                        dst_v.at[pl.ds(0, BASECH)])

        @pl.when(wid < EXTRA)
        def _():
            pltpu.sync_copy(ei3.at[0, BASECH * NW + wid], src_v.at[BASECH])
            pltpu.sync_copy(ei3.at[1, BASECH * NW + wid], dst_v.at[BASECH])

        if with_count:
            pltpu.sync_copy(zcnt.at[pl.ds(r0, RPT)], sh_cnt.at[pl.ds(r0, RPT)])
            pltpu.sync_copy(ones, ones_v)
        plsc.subcore_barrier()

        bufs = ((rows0, g0, s0, c0), (rows1, g1, s1, c1),
                (rows2, g2, s2, c2))
        NB = len(bufs)

        pltpu.async_copy(table.at[src_v.at[0]], rows0, g0)
        pltpu.async_copy(table.at[src_v.at[1]], rows1, g1)
        pltpu.async_copy(table.at[src_v.at[2]], rows2, g2)

        def pair_body(i, carry):
            for b, (buf, gsem, ssem, csem) in enumerate(bufs):
                c = NB * i + b
                pltpu.make_async_copy(table.at[src_v.at[c]], buf, gsem).wait()
                sc = pltpu.async_copy(buf, sh_agg.at[dst_v.at[c]], ssem,
                                      add=True)
                if with_count:
                    cc = pltpu.async_copy(ones_v, sh_cnt.at[dst_v.at[c]],
                                          csem, add=True)
                sc.wait()

                @pl.when(i < BASECH // NB - 1)
                def _():
                    pltpu.async_copy(table.at[src_v.at[c + NB]], buf, gsem)

                if with_count:
                    cc.wait()
            return carry

        lax.fori_loop(0, BASECH // NB, pair_body, 0)

        @pl.when(wid < EXTRA)
        def _():
            pltpu.async_copy(table.at[src_v.at[BASECH]], rows0, g0).wait()
            sc = pltpu.async_copy(rows0, sh_agg.at[dst_v.at[BASECH]], s0,
                                  add=True)
            if with_count:
                pltpu.async_copy(ones_v, sh_cnt.at[dst_v.at[BASECH]], c0,
                                 add=True).wait()
            sc.wait()

        plsc.subcore_barrier()
        pltpu.sync_copy(sh_agg.at[pl.ds(r0, RPT)],
                        agg_out.at[cid, pl.ds(r0, RPT)])
        if with_count:
            pltpu.sync_copy(sh_cnt.at[pl.ds(r0, RPT)],
                            cnt_out.at[cid, pl.ds(r0, RPT)])

    if with_count:
        def body_wc(table, ei3, zagg, zcnt, ones, agg_out, cnt_out,
                    src_v, dst_v, rows0, rows1, rows2, sh_agg, g0, g1, g2,
                    s0, s1, s2, ones_v, sh_cnt, c0, c1, c2):
            body(table, ei3, zagg, zcnt, ones, agg_out, cnt_out,
                 src_v, dst_v, rows0, rows1, rows2, sh_agg, g0, g1, g2,
                 s0, s1, s2, ones_v, sh_cnt, c0, c1, c2)
        fn = pl.kernel(body_wc, out_type=out_type, mesh=mesh,
                       scratch_types=scratch,
                       compiler_params=pltpu.CompilerParams(
                           use_tc_tiling_on_sc=False))

        def run(table, ei3, zagg, zcnt, ones):
            return fn(table, ei3, zagg, zcnt, ones)
    else:
        def body_nc(table, ei3, zagg, agg_out,
                    src_v, dst_v, rows0, rows1, rows2, sh_agg, g0, g1, g2,
                    s0, s1, s2):
            body(table, ei3, zagg, None, None, agg_out, None,
                 src_v, dst_v, rows0, rows1, rows2, sh_agg, g0, g1, g2,
                 s0, s1, s2)
        fn = pl.kernel(body_nc, out_type=out_type, mesh=mesh,
                       scratch_types=scratch,
                       compiler_params=pltpu.CompilerParams(
                           use_tc_tiling_on_sc=False))

        def run(table, ei3, zagg, zcnt, ones):
            return fn(table, ei3, zagg)
    return run


_seg_count = _make_seg(True)
_seg_plain = _make_seg(False)


def _act_body(agg_ref, cnt_ref, xr_ref, b1_ref, w_ref, h_ref, hr_ref):
    aggs = agg_ref[0, :P] + agg_ref[1, :P]
    cnts = cnt_ref[0, :P] + cnt_ref[1, :P]
    inv = 1.0 / jnp.maximum(cnts, 1.0)
    h = jnp.maximum(aggs * inv + b1_ref[...] + xr_ref[...], 0.0)
    h_ref[...] = h
    hr_ref[...] = jnp.dot(h, w_ref[...], preferred_element_type=jnp.float32)


def _act(aggp, cntp, xrp, b1t, w2rbig):
    return pl.pallas_call(
        _act_body,
        out_shape=[
            jax.ShapeDtypeStruct((P, PL), jnp.float32),
            jax.ShapeDtypeStruct((P, PC), jnp.float32),
        ],
    )(aggp, cntp, xrp, b1t, w2rbig)


def _out_body(agg_ref, cnt_ref, hr_ref, b2_ref, w_ref, s_ref, o_ref):
    aggs = agg_ref[0, :P] + agg_ref[1, :P]
    cnts = cnt_ref[0, :P] + cnt_ref[1, :P]
    mean2 = aggs * (1.0 / jnp.maximum(cnts, 1.0))
    o = (jnp.dot(mean2, w_ref[...], preferred_element_type=jnp.float32)
         + b2_ref[...] + hr_ref[...])
    m = jnp.max(o, axis=1, keepdims=True)
    e = jnp.exp(o - m)
    s = jnp.dot(e, s_ref[...], preferred_element_type=jnp.float32)
    o_ref[...] = o - m - jnp.log(s)


def _outk(agg2p, cntp, hr2p, b2t, w2lbig, smat):
    return pl.pallas_call(
        _out_body,
        out_shape=jax.ShapeDtypeStruct((P, PC), jnp.float32),
    )(agg2p, cntp, hr2p, b2t, w2lbig, smat)


@jax.jit
def kernel(x, edge_index, W1l, b1, W1r, W2l, b2, W2r):
    ei3 = edge_index.reshape(2, TOTCH, CHUNK)
    zagg = jnp.zeros((NPAD, H), jnp.float32)
    zcnt = jnp.zeros((NPAD, H), jnp.float32)
    ones = jnp.ones((CHUNK, H), jnp.float32)
    eye8 = jnp.eye(8, dtype=jnp.float32)
    x1024 = x.reshape(P, 8 * D)
    w1big = jnp.concatenate(
        [jnp.kron(eye8, W1l.T), jnp.kron(eye8, W1r.T)], axis=1)
    w2rbig = jnp.kron(eye8, W2r.T)
    w2lbig = jnp.kron(eye8, W2l.T)
    smat = jnp.kron(eye8, jnp.ones((C, C), jnp.float32))
    b1t = jnp.tile(b1, 8).reshape(1, PL)
    b2t = jnp.tile(b2, 8).reshape(1, PC)

    y1p, xrp = _mm1(x1024, w1big)
    agg1, cnt = _seg_count(y1p.reshape(N, H), ei3, zagg, zcnt, ones)
    hp, hr2p = _act(agg1.reshape(NC, NPAD // 8, PL),
                    cnt.reshape(NC, NPAD // 8, PL), xrp, b1t, w2rbig)

    (agg2,) = _seg_plain(hp.reshape(N, H), ei3, zagg, zcnt, ones)
    outp = _outk(agg2.reshape(NC, NPAD // 8, PL),
                 cnt.reshape(NC, NPAD // 8, PL), hr2p, b2t, w2lbig, smat)
    return outp.reshape(N, C)


# speedup vs baseline: 34.6794x; 1.1976x over previous
import jax
import jax.numpy as jnp
from jax import lax
from jax.experimental import pallas as pl
from jax.experimental.pallas import tpu as pltpu
from jax.experimental.pallas import tpu_sc as plsc

N = 10000
D = 128
H = 16
C = 40
E = 320000

NPAD = 10240
NC = 2
NS = 16
NW = NC * NS
CHUNK = 128
TOTCH = E // CHUNK
BASECH = TOTCH // NW
EXTRA = TOTCH - BASECH * NW
RPT = NPAD // NS


P = N // 8
PL = 8 * H
PC = 8 * C


def _mm1_body(x_ref, wb_ref, y1_ref, xr_ref):
    o = jnp.dot(x_ref[...], wb_ref[...], preferred_element_type=jnp.float32)
    y1_ref[...] = o[:, :PL]
    xr_ref[...] = o[:, PL:]


def _mm1(x1024, w1big):
    return pl.pallas_call(
        _mm1_body,
        out_shape=[
            jax.ShapeDtypeStruct((P, PL), jnp.float32),
            jax.ShapeDtypeStruct((P, PL), jnp.float32),
        ],
    )(x1024, w1big)


NB = 6


def _make_seg(with_count):
    mesh = plsc.VectorSubcoreMesh(core_axis_name="c", subcore_axis_name="s")
    out_type = [jax.ShapeDtypeStruct((NC, NPAD, H), jnp.float32)]
    scratch = (
        [pltpu.VMEM((BASECH + 1, CHUNK), jnp.int32)] * 2
        + [pltpu.VMEM((CHUNK, H), jnp.float32)] * NB
        + [pltpu.VMEM_SHARED((NPAD, H), jnp.float32)]
        + [pltpu.SemaphoreType.DMA] * (2 * NB)
    )
    if with_count:
        out_type.append(jax.ShapeDtypeStruct((NC, NPAD, H), jnp.float32))
        scratch += (
            [pltpu.VMEM((CHUNK, H), jnp.float32)]
            + [pltpu.VMEM_SHARED((NPAD, H), jnp.float32)]
            + [pltpu.SemaphoreType.DMA] * NB
        )

    def body(table, ei3, zagg, zcnt, ones, agg_out, cnt_out, scr):
        src_v, dst_v = scr[0], scr[1]
        rows = scr[2:2 + NB]
        sh_agg = scr[2 + NB]
        gsems = scr[3 + NB:3 + 2 * NB]
        ssems = scr[3 + 2 * NB:3 + 3 * NB]
        if with_count:
            ones_v = scr[3 + 3 * NB]
            sh_cnt = scr[4 + 3 * NB]
            csems = scr[5 + 3 * NB:5 + 4 * NB]
        else:
            ones_v = sh_cnt = None
            csems = [None] * NB
        cid = lax.axis_index("c")
        sid = lax.axis_index("s")
        wid = sid * NC + cid
        r0 = sid * RPT
        pltpu.sync_copy(zagg.at[pl.ds(r0, RPT)], sh_agg.at[pl.ds(r0, RPT)])
        base = wid * BASECH
        pltpu.sync_copy(ei3.at[0, pl.ds(base, BASECH)],
                        src_v.at[pl.ds(0, BASECH)])
        pltpu.sync_copy(ei3.at[1, pl.ds(base, BASECH)],
                        dst_v.at[pl.ds(0, BASECH)])

        @pl.when(wid < EXTRA)
        def _():
            pltpu.sync_copy(ei3.at[0, BASECH * NW + wid], src_v.at[BASECH])
            pltpu.sync_copy(ei3.at[1, BASECH * NW + wid], dst_v.at[BASECH])

        if with_count:
            pltpu.sync_copy(zcnt.at[pl.ds(r0, RPT)], sh_cnt.at[pl.ds(r0, RPT)])
            pltpu.sync_copy(ones, ones_v)
        plsc.subcore_barrier()

        for b in range(NB):
            pltpu.async_copy(table.at[src_v.at[b]], rows[b], gsems[b])

        def ring_body(i, carry):
            for b in range(NB):
                c = NB * i + b
                pltpu.make_async_copy(table.at[src_v.at[c]], rows[b],
                                      gsems[b]).wait()
                sc = pltpu.async_copy(rows[b], sh_agg.at[dst_v.at[c]],
                                      ssems[b], add=True)
                if with_count:
                    cc = pltpu.async_copy(ones_v, sh_cnt.at[dst_v.at[c]],
                                          csems[b], add=True)
                sc.wait()

                @pl.when(i < BASECH // NB - 1)
                def _():
                    pltpu.async_copy(table.at[src_v.at[c + NB]], rows[b],
                                     gsems[b])

                if with_count:
                    cc.wait()
            return carry

        lax.fori_loop(0, BASECH // NB, ring_body, 0)

        @pl.when(wid < EXTRA)
        def _():
            pltpu.async_copy(table.at[src_v.at[BASECH]], rows[0],
                             gsems[0]).wait()
            sc = pltpu.async_copy(rows[0], sh_agg.at[dst_v.at[BASECH]],
                                  ssems[0], add=True)
            if with_count:
                pltpu.async_copy(ones_v, sh_cnt.at[dst_v.at[BASECH]],
                                 csems[0], add=True).wait()
            sc.wait()

        plsc.subcore_barrier()
        pltpu.sync_copy(sh_agg.at[pl.ds(r0, RPT)],
                        agg_out.at[cid, pl.ds(r0, RPT)])
        if with_count:
            pltpu.sync_copy(sh_cnt.at[pl.ds(r0, RPT)],
                            cnt_out.at[cid, pl.ds(r0, RPT)])

    if with_count:
        def entry(table, ei3, zagg, zcnt, ones, agg_out, cnt_out, *scr):
            body(table, ei3, zagg, zcnt, ones, agg_out, cnt_out, scr)
    else:
        def entry(table, ei3, zagg, agg_out, *scr):
            body(table, ei3, zagg, None, None, agg_out, None, scr)
    fn = pl.kernel(entry, out_type=out_type, mesh=mesh,
                   scratch_types=scratch,
                   compiler_params=pltpu.CompilerParams(
                       use_tc_tiling_on_sc=False))

    if with_count:
        def run(table, ei3, zagg, zcnt, ones):
            return fn(table, ei3, zagg, zcnt, ones)
    else:
        def run(table, ei3, zagg, zcnt, ones):
            return fn(table, ei3, zagg)
    return run


_seg_count = _make_seg(True)
_seg_plain = _make_seg(False)


def _act_body(agg_ref, cnt_ref, xr_ref, b1_ref, w_ref, h_ref, hr_ref):
    aggs = agg_ref[0, :P] + agg_ref[1, :P]
    cnts = cnt_ref[0, :P] + cnt_ref[1, :P]
    inv = 1.0 / jnp.maximum(cnts, 1.0)
    h = jnp.maximum(aggs * inv + b1_ref[...] + xr_ref[...], 0.0)
    h_ref[...] = h
    hr_ref[...] = jnp.dot(h, w_ref[...], preferred_element_type=jnp.float32)


def _act(aggp, cntp, xrp, b1t, w2rbig):
    return pl.pallas_call(
        _act_body,
        out_shape=[
            jax.ShapeDtypeStruct((P, PL), jnp.float32),
            jax.ShapeDtypeStruct((P, PC), jnp.float32),
        ],
    )(aggp, cntp, xrp, b1t, w2rbig)


def _out_body(agg_ref, cnt_ref, hr_ref, b2_ref, w_ref, s_ref, o_ref):
    aggs = agg_ref[0, :P] + agg_ref[1, :P]
    cnts = cnt_ref[0, :P] + cnt_ref[1, :P]
    mean2 = aggs * (1.0 / jnp.maximum(cnts, 1.0))
    o = (jnp.dot(mean2, w_ref[...], preferred_element_type=jnp.float32)
         + b2_ref[...] + hr_ref[...])
    m = jnp.max(o, axis=1, keepdims=True)
    e = jnp.exp(o - m)
    s = jnp.dot(e, s_ref[...], preferred_element_type=jnp.float32)
    o_ref[...] = o - m - jnp.log(s)


def _outk(agg2p, cntp, hr2p, b2t, w2lbig, smat):
    return pl.pallas_call(
        _out_body,
        out_shape=jax.ShapeDtypeStruct((P, PC), jnp.float32),
    )(agg2p, cntp, hr2p, b2t, w2lbig, smat)


@jax.jit
def kernel(x, edge_index, W1l, b1, W1r, W2l, b2, W2r):
    ei3 = edge_index.reshape(2, TOTCH, CHUNK)
    zagg = jnp.zeros((NPAD, H), jnp.float32)
    zcnt = jnp.zeros((NPAD, H), jnp.float32)
    ones = jnp.ones((CHUNK, H), jnp.float32)
    eye8 = jnp.eye(8, dtype=jnp.float32)
    x1024 = x.reshape(P, 8 * D)
    w1big = jnp.concatenate(
        [jnp.kron(eye8, W1l.T), jnp.kron(eye8, W1r.T)], axis=1)
    w2rbig = jnp.kron(eye8, W2r.T)
    w2lbig = jnp.kron(eye8, W2l.T)
    smat = jnp.kron(eye8, jnp.ones((C, C), jnp.float32))
    b1t = jnp.tile(b1, 8).reshape(1, PL)
    b2t = jnp.tile(b2, 8).reshape(1, PC)

    y1p, xrp = _mm1(x1024, w1big)
    agg1, cnt = _seg_count(y1p.reshape(N, H), ei3, zagg, zcnt, ones)
    hp, hr2p = _act(agg1.reshape(NC, NPAD // 8, PL),
                    cnt.reshape(NC, NPAD // 8, PL), xrp, b1t, w2rbig)

    (agg2,) = _seg_plain(hp.reshape(N, H), ei3, zagg, zcnt, ones)
    outp = _outk(agg2.reshape(NC, NPAD // 8, PL),
                 cnt.reshape(NC, NPAD // 8, PL), hr2p, b2t, w2lbig, smat)
    return outp.reshape(N, C)


# speedup vs baseline: 34.7290x; 1.0014x over previous
import jax
import jax.numpy as jnp
from jax import lax
from jax.experimental import pallas as pl
from jax.experimental.pallas import tpu as pltpu
from jax.experimental.pallas import tpu_sc as plsc

N = 10000
D = 128
H = 16
C = 40
E = 320000

NPAD = 10240
NC = 2
NS = 16
NW = NC * NS
CHUNK = 128
TOTCH = E // CHUNK
BASECH = TOTCH // NW
EXTRA = TOTCH - BASECH * NW
RPT = NPAD // NS


P = N // 8
PL = 8 * H
PC = 8 * C


def _mm1_body(x_ref, wb_ref, y1_ref, xr_ref):
    o = jnp.dot(x_ref[...], wb_ref[...], preferred_element_type=jnp.float32)
    y1_ref[...] = o[:, :PL]
    xr_ref[...] = o[:, PL:]


def _mm1(x1024, w1big):
    return pl.pallas_call(
        _mm1_body,
        out_shape=[
            jax.ShapeDtypeStruct((P, PL), jnp.float32),
            jax.ShapeDtypeStruct((P, PL), jnp.float32),
        ],
    )(x1024, w1big)


NB = 6


def _make_seg(with_count):
    mesh = plsc.VectorSubcoreMesh(core_axis_name="c", subcore_axis_name="s")
    out_type = [jax.ShapeDtypeStruct((NC, NPAD, H), jnp.float32)]
    scratch = (
        [pltpu.VMEM((BASECH + 1, CHUNK), jnp.int32)] * 2
        + [pltpu.VMEM((CHUNK, H), jnp.float32)] * NB
        + [pltpu.VMEM_SHARED((NPAD, H), jnp.float32)]
        + [pltpu.SemaphoreType.DMA] * (2 * NB)
    )
    if with_count:
        out_type.append(jax.ShapeDtypeStruct((NC, NPAD, H), jnp.float32))
        scratch += (
            [pltpu.VMEM((CHUNK, H), jnp.float32)]
            + [pltpu.VMEM_SHARED((NPAD, H), jnp.float32)]
            + [pltpu.SemaphoreType.DMA] * NB
        )

    def body(table, ei3, zagg, zcnt, ones, agg_out, cnt_out, scr):
        src_v, dst_v = scr[0], scr[1]
        rows = scr[2:2 + NB]
        sh_agg = scr[2 + NB]
        gsems = scr[3 + NB:3 + 2 * NB]
        ssems = scr[3 + 2 * NB:3 + 3 * NB]
        if with_count:
            ones_v = scr[3 + 3 * NB]
            sh_cnt = scr[4 + 3 * NB]
            csems = scr[5 + 3 * NB:5 + 4 * NB]
        else:
            ones_v = sh_cnt = None
            csems = [None] * NB
        cid = lax.axis_index("c")
        sid = lax.axis_index("s")
        wid = sid * NC + cid
        r0 = sid * RPT
        pltpu.sync_copy(zagg.at[pl.ds(r0, RPT)], sh_agg.at[pl.ds(r0, RPT)])
        base = wid * BASECH
        pltpu.sync_copy(ei3.at[0, pl.ds(base, BASECH)],
                        src_v.at[pl.ds(0, BASECH)])
        pltpu.sync_copy(ei3.at[1, pl.ds(base, BASECH)],
                        dst_v.at[pl.ds(0, BASECH)])

        @pl.when(wid < EXTRA)
        def _():
            pltpu.sync_copy(ei3.at[0, BASECH * NW + wid], src_v.at[BASECH])
            pltpu.sync_copy(ei3.at[1, BASECH * NW + wid], dst_v.at[BASECH])

        if with_count:
            pltpu.sync_copy(zcnt.at[pl.ds(r0, RPT)], sh_cnt.at[pl.ds(r0, RPT)])
            pltpu.sync_copy(ones, ones_v)
        plsc.subcore_barrier()

        for b in range(NB):
            pltpu.async_copy(table.at[src_v.at[b]], rows[b], gsems[b])

        def ring_body(i, carry):
            for b in range(NB):
                c = NB * i + b
                bp = (b - 1) % NB
                cp = c - 1
                pltpu.make_async_copy(table.at[src_v.at[c]], rows[b],
                                      gsems[b]).wait()
                pltpu.async_copy(rows[b], sh_agg.at[dst_v.at[c]],
                                 ssems[b], add=True)
                if with_count:
                    pltpu.async_copy(ones_v, sh_cnt.at[dst_v.at[c]],
                                     csems[b], add=True)

                @pl.when(cp >= 0)
                def _():
                    pltpu.make_async_copy(rows[bp], sh_agg.at[dst_v.at[cp]],
                                          ssems[bp]).wait()

                    @pl.when(cp + NB < BASECH)
                    def _():
                        pltpu.async_copy(table.at[src_v.at[cp + NB]],
                                         rows[bp], gsems[bp])

                    if with_count:
                        pltpu.make_async_copy(ones_v,
                                              sh_cnt.at[dst_v.at[cp]],
                                              csems[bp]).wait()
            return carry

        lax.fori_loop(0, BASECH // NB, ring_body, 0)

        lastb = (BASECH - 1) % NB
        pltpu.make_async_copy(rows[lastb], sh_agg.at[dst_v.at[BASECH - 1]],
                              ssems[lastb]).wait()
        if with_count:
            pltpu.make_async_copy(ones_v, sh_cnt.at[dst_v.at[BASECH - 1]],
                                  csems[lastb]).wait()

        @pl.when(wid < EXTRA)
        def _():
            pltpu.async_copy(table.at[src_v.at[BASECH]], rows[0],
                             gsems[0]).wait()
            sc = pltpu.async_copy(rows[0], sh_agg.at[dst_v.at[BASECH]],
                                  ssems[0], add=True)
            if with_count:
                pltpu.async_copy(ones_v, sh_cnt.at[dst_v.at[BASECH]],
                                 csems[0], add=True).wait()
            sc.wait()

        plsc.subcore_barrier()
        pltpu.sync_copy(sh_agg.at[pl.ds(r0, RPT)],
                        agg_out.at[cid, pl.ds(r0, RPT)])
        if with_count:
            pltpu.sync_copy(sh_cnt.at[pl.ds(r0, RPT)],
                            cnt_out.at[cid, pl.ds(r0, RPT)])

    if with_count:
        def entry(table, ei3, zagg, zcnt, ones, agg_out, cnt_out, *scr):
            body(table, ei3, zagg, zcnt, ones, agg_out, cnt_out, scr)
    else:
        def entry(table, ei3, zagg, agg_out, *scr):
            body(table, ei3, zagg, None, None, agg_out, None, scr)
    fn = pl.kernel(entry, out_type=out_type, mesh=mesh,
                   scratch_types=scratch,
                   compiler_params=pltpu.CompilerParams(
                       use_tc_tiling_on_sc=False))

    if with_count:
        def run(table, ei3, zagg, zcnt, ones):
            return fn(table, ei3, zagg, zcnt, ones)
    else:
        def run(table, ei3, zagg, zcnt, ones):
            return fn(table, ei3, zagg)
    return run


_seg_count = _make_seg(True)
_seg_plain = _make_seg(False)


def _act_body(agg_ref, cnt_ref, xr_ref, b1_ref, w_ref, h_ref, hr_ref):
    aggs = agg_ref[0, :P] + agg_ref[1, :P]
    cnts = cnt_ref[0, :P] + cnt_ref[1, :P]
    inv = 1.0 / jnp.maximum(cnts, 1.0)
    h = jnp.maximum(aggs * inv + b1_ref[...] + xr_ref[...], 0.0)
    h_ref[...] = h
    hr_ref[...] = jnp.dot(h, w_ref[...], preferred_element_type=jnp.float32)


def _act(aggp, cntp, xrp, b1t, w2rbig):
    return pl.pallas_call(
        _act_body,
        out_shape=[
            jax.ShapeDtypeStruct((P, PL), jnp.float32),
            jax.ShapeDtypeStruct((P, PC), jnp.float32),
        ],
    )(aggp, cntp, xrp, b1t, w2rbig)


def _out_body(agg_ref, cnt_ref, hr_ref, b2_ref, w_ref, s_ref, o_ref):
    aggs = agg_ref[0, :P] + agg_ref[1, :P]
    cnts = cnt_ref[0, :P] + cnt_ref[1, :P]
    mean2 = aggs * (1.0 / jnp.maximum(cnts, 1.0))
    o = (jnp.dot(mean2, w_ref[...], preferred_element_type=jnp.float32)
         + b2_ref[...] + hr_ref[...])
    m = jnp.max(o, axis=1, keepdims=True)
    e = jnp.exp(o - m)
    s = jnp.dot(e, s_ref[...], preferred_element_type=jnp.float32)
    o_ref[...] = o - m - jnp.log(s)


def _outk(agg2p, cntp, hr2p, b2t, w2lbig, smat):
    return pl.pallas_call(
        _out_body,
        out_shape=jax.ShapeDtypeStruct((P, PC), jnp.float32),
    )(agg2p, cntp, hr2p, b2t, w2lbig, smat)


@jax.jit
def kernel(x, edge_index, W1l, b1, W1r, W2l, b2, W2r):
    ei3 = edge_index.reshape(2, TOTCH, CHUNK)
    zagg = jnp.zeros((NPAD, H), jnp.float32)
    zcnt = jnp.zeros((NPAD, H), jnp.float32)
    ones = jnp.ones((CHUNK, H), jnp.float32)
    eye8 = jnp.eye(8, dtype=jnp.float32)
    x1024 = x.reshape(P, 8 * D)
    w1big = jnp.concatenate(
        [jnp.kron(eye8, W1l.T), jnp.kron(eye8, W1r.T)], axis=1)
    w2rbig = jnp.kron(eye8, W2r.T)
    w2lbig = jnp.kron(eye8, W2l.T)
    smat = jnp.kron(eye8, jnp.ones((C, C), jnp.float32))
    b1t = jnp.tile(b1, 8).reshape(1, PL)
    b2t = jnp.tile(b2, 8).reshape(1, PC)

    y1p, xrp = _mm1(x1024, w1big)
    agg1, cnt = _seg_count(y1p.reshape(N, H), ei3, zagg, zcnt, ones)
    hp, hr2p = _act(agg1.reshape(NC, NPAD // 8, PL),
                    cnt.reshape(NC, NPAD // 8, PL), xrp, b1t, w2rbig)

    (agg2,) = _seg_plain(hp.reshape(N, H), ei3, zagg, zcnt, ones)
    outp = _outk(agg2.reshape(NC, NPAD // 8, PL),
                 cnt.reshape(NC, NPAD // 8, PL), hr2p, b2t, w2lbig, smat)
    return outp.reshape(N, C)
